# baseline (speedup 1.0000x reference)
import functools

import jax
import jax.numpy as jnp
from jax import lax
from jax.experimental import pallas as pl
from jax.experimental.pallas import tpu as pltpu
from jax.experimental.pallas import tpu_sc as plsc

NC = 2
NS = 16
CHUNK = 80
NBUF = 2
IDXN = 4
INNER = 4


def _sc_aggregate(n_pad, e_pad, d):
  nw = NC * NS
  edges_per_worker = e_pad // nw
  nchunks = edges_per_worker // CHUNK
  nouter = nchunks // INNER
  rows_per_tile = n_pad // NS
  zch = rows_per_tile // 8
  mesh = plsc.VectorSubcoreMesh(
      core_axis_name="c", subcore_axis_name="s", num_cores=NC, num_subcores=NS)

  @functools.partial(
      pl.kernel,
      mesh=mesh,
      out_type=[
          jax.ShapeDtypeStruct((NC, n_pad, d), jnp.float32),
          jax.ShapeDtypeStruct((NC, NS, n_pad), jnp.float32),
      ],
      scratch_types=[
          [pltpu.VMEM((2, CHUNK), jnp.int32) for _ in range(IDXN)],
          [pltpu.VMEM((CHUNK, d), jnp.float32) for _ in range(NBUF)],
          pltpu.VMEM((n_pad,), jnp.float32),
          pltpu.VMEM_SHARED((n_pad, d), jnp.float32),
          [pltpu.SemaphoreType.DMA for _ in range(IDXN)],
          [pltpu.SemaphoreType.DMA for _ in range(NBUF)],
          [pltpu.SemaphoreType.DMA for _ in range(NBUF)],
      ],
  )
  def agg_kernel(x_hbm, pair_hbm, zrow_hbm, zcnt_hbm, agg_out, cnt_out,
                 ring, bufs, hist, agg_sh, isems, gsems, ssems):
    cid = lax.axis_index("c")
    sid = lax.axis_index("s")
    w = cid * NS + sid
    r0 = sid * rows_per_tile
    pltpu.sync_copy(zrow_hbm, bufs[0])
    for q in range(8):
      pltpu.sync_copy(bufs[0], agg_sh.at[pl.ds(r0 + q * zch, zch)])
    pltpu.sync_copy(zcnt_hbm, hist)
    plsc.subcore_barrier()

    lanes = lax.iota(jnp.int32, 16)

    def hist_chunk(s):
      def edge_group(j, c):
        dv = ring[s][1, pl.ds(j * 16, 16)]
        for k in range(16):
          dd = dv[k]
          base = pl.multiple_of((dd // 16) * 16, 16)
          one = jnp.where(lanes == dd - base, 1.0, 0.0)
          hist[pl.ds(base, 16)] = hist[pl.ds(base, 16)] + one
        return c

      lax.fori_loop(0, CHUNK // 16, edge_group, 0)

    def idx_fetch(g, s):
      pltpu.async_copy(pair_hbm.at[w, g], ring[s], isems[s])

    def idx_wait(s):
      pltpu.make_async_copy(pair_hbm.at[w, 0], ring[s], isems[s]).wait()

    def gather(s, b):
      pltpu.async_copy(x_hbm.at[ring[s].at[0]], bufs[b], gsems[b])

    def gather_wait(s, b):
      pltpu.make_async_copy(x_hbm.at[ring[s].at[0]], bufs[b], gsems[b]).wait()

    def scatter(s, b):
      pltpu.async_copy(bufs[b], agg_sh.at[ring[s].at[1]], ssems[b], add=True)

    def scatter_wait(s, b):
      pltpu.make_async_copy(bufs[b], agg_sh.at[ring[s].at[1]],
                            ssems[b]).wait()

    for s in range(IDXN - 1):
      idx_fetch(s, s)
    idx_wait(0)
    gather(0, 0)

    def outer(p, carry):
      for b in range(INNER):
        g = p * INNER + b
        s, s1, s3 = b % IDXN, (b + 1) % IDXN, (b + 3) % IDXN
        rb, rb1 = b % NBUF, (b + 1) % NBUF
        if b == 0:
          @pl.when(p >= 1)
          def _():
            scatter_wait(s3, rb1)
        else:
          scatter_wait(s3, rb1)
        if b == 0:
          idx_fetch(g + 3, s3)
        else:
          @pl.when(p < nouter - 1)
          def _():
            idx_fetch(g + 3, s3)
        if b == INNER - 1:
          @pl.when(p < nouter - 1)
          def _():
            idx_wait(s1)
            gather(s1, rb1)
        else:
          idx_wait(s1)
          gather(s1, rb1)
        hist_chunk(s)
        gather_wait(s, rb)
        scatter(s, rb)
      return carry

    lax.fori_loop(0, nouter, outer, 0)
    scatter_wait((nchunks - 1) % IDXN, (nchunks - 1) % NBUF)
    pltpu.sync_copy(hist, cnt_out.at[cid, sid])
    plsc.subcore_barrier()

    for q in range(8):
      rq = r0 + q * zch
      pltpu.sync_copy(agg_sh.at[pl.ds(rq, zch)], bufs[0])
      pltpu.sync_copy(bufs[0], agg_out.at[cid, pl.ds(rq, zch)])

  return agg_kernel


def _tc_body(agg_ref, cnt_ref, x_ref, wn_ref, ws_ref, b_ref, o_ref):
  blk, d = x_ref.shape
  nsub = blk // d
  agg = agg_ref[0] + agg_ref[1]
  cnt = jnp.sum(cnt_ref[...], axis=(0, 1)).reshape(nsub, d)
  recip = 1.0 / jnp.maximum(cnt, 1.0)
  dn = (((1,), (1,)), ((), ()))
  aggw = lax.dot_general(agg, wn_ref[...], dn,
                         preferred_element_type=jnp.float32)
  hs = lax.dot_general(x_ref[...], ws_ref[...], dn,
                       preferred_element_type=jnp.float32)
  ii = lax.broadcasted_iota(jnp.int32, (d, d), 0)
  jj = lax.broadcasted_iota(jnp.int32, (d, d), 1)
  eye = jnp.where(ii == jj, 1.0, 0.0)
  for a in range(nsub):
    dmat = eye * jnp.broadcast_to(recip[a:a + 1, :], (d, d))
    hn_a = lax.dot_general(dmat, aggw[a * d:(a + 1) * d, :],
                           (((1,), (0,)), ((), ())),
                           preferred_element_type=jnp.float32)
    h = hn_a + hs[a * d:(a + 1) * d, :] + b_ref[...]
    h = jnp.where(jnp.isfinite(h), h, 0.0)
    norm = jnp.sqrt(jnp.sum(h * h, axis=1, keepdims=True))
    o_ref[pl.ds(a * d, d), :] = h / jnp.maximum(norm, 1e-12)


def kernel(x, edge_index, W_neigh, W_self, bias):
  n, d = x.shape
  e = edge_index.shape[1]
  nw = NC * NS
  n_pad = 10240 if n == 10000 else ((n + 16 * d - 1) // (16 * d)) * (16 * d)
  egrain = nw * CHUNK * INNER
  e_pad = ((e + egrain - 1) // egrain) * egrain
  pad_rows = n_pad - n

  src = edge_index[0]
  dst = edge_index[1]
  if e_pad != e:
    pad_idx = (jnp.arange(e_pad - e, dtype=jnp.int32) % pad_rows) + n
    src = jnp.concatenate([src, pad_idx])
    dst = jnp.concatenate([dst, pad_idx])
  nchunks = e_pad // nw // CHUNK
  pairs = jnp.stack([src.reshape(nw, nchunks, CHUNK),
                     dst.reshape(nw, nchunks, CHUNK)], axis=2)
  x_pad = jnp.concatenate([x, jnp.zeros((pad_rows, d), x.dtype)])

  zrow = jnp.zeros((CHUNK, d), jnp.float32)
  zcnt = jnp.zeros((n_pad,), jnp.float32)

  agg_parts, cnt_parts = _sc_aggregate(n_pad, e_pad, d)(
      x_pad, pairs, zrow, zcnt)

  blk = n_pad // 4
  nsub = blk // d
  h = pl.pallas_call(
      _tc_body,
      grid=(n_pad // blk,),
      in_specs=[
          pl.BlockSpec((NC, blk, d), lambda i: (0, i, 0)),
          pl.BlockSpec((NC, NS, blk), lambda i: (0, 0, i)),
          pl.BlockSpec((blk, d), lambda i: (i, 0)),
          pl.BlockSpec((d, d), lambda i: (0, 0)),
          pl.BlockSpec((d, d), lambda i: (0, 0)),
          pl.BlockSpec((1, d), lambda i: (0, 0)),
      ],
      out_specs=pl.BlockSpec((blk, d), lambda i: (i, 0)),
      out_shape=jax.ShapeDtypeStruct((n_pad, d), jnp.float32),
  )(agg_parts, cnt_parts, x_pad, W_neigh, W_self, bias.reshape(1, d))
  return h[:n]


# speedup vs baseline: 1.0720x; 1.0720x over previous
import functools

import jax
import jax.numpy as jnp
from jax import lax
from jax.experimental import pallas as pl
from jax.experimental.pallas import tpu as pltpu
from jax.experimental.pallas import tpu_sc as plsc

NC = 2
NS = 16
CHUNK = 80
NBUF = 2
IDXN = 4
INNER = 4


def _sc_aggregate(n_pad, e_pad, d):
  nw = NC * NS
  edges_per_worker = e_pad // nw
  nchunks = edges_per_worker // CHUNK
  nouter = nchunks // INNER
  rows_per_tile = n_pad // NS
  zch = rows_per_tile // 8
  mesh = plsc.VectorSubcoreMesh(
      core_axis_name="c", subcore_axis_name="s", num_cores=NC, num_subcores=NS)

  @functools.partial(
      pl.kernel,
      mesh=mesh,
      out_type=[
          jax.ShapeDtypeStruct((NC, n_pad, d), jnp.float32),
          jax.ShapeDtypeStruct((NC, NS, n_pad), jnp.float32),
      ],
      scratch_types=[
          [pltpu.VMEM((2, CHUNK), jnp.int32) for _ in range(IDXN)],
          [pltpu.VMEM((CHUNK, d), jnp.float32) for _ in range(NBUF)],
          pltpu.VMEM((n_pad,), jnp.float32),
          pltpu.VMEM_SHARED((n_pad, d), jnp.float32),
          [pltpu.SemaphoreType.DMA for _ in range(IDXN)],
          [pltpu.SemaphoreType.DMA for _ in range(NBUF)],
          [pltpu.SemaphoreType.DMA for _ in range(NBUF)],
      ],
  )
  def agg_kernel(x_hbm, pair_hbm, zrow_hbm, zcnt_hbm, agg_out, cnt_out,
                 ring, bufs, hist, agg_sh, isems, gsems, ssems):
    cid = lax.axis_index("c")
    sid = lax.axis_index("s")
    w = cid * NS + sid
    r0 = sid * rows_per_tile
    pltpu.sync_copy(zrow_hbm, bufs[0])
    for q in range(8):
      pltpu.sync_copy(bufs[0], agg_sh.at[pl.ds(r0 + q * zch, zch)])
    pltpu.sync_copy(zcnt_hbm, hist)
    plsc.subcore_barrier()

    lanes = lax.iota(jnp.int32, 16)

    def hist_chunk(s):
      def edge_group(j, c):
        dv = ring[s][1, pl.ds(j * 16, 16)]
        for k in range(16):
          dd = dv[k]
          base = pl.multiple_of((dd // 16) * 16, 16)
          one = jnp.where(lanes == dd - base, 1.0, 0.0)
          hist[pl.ds(base, 16)] = hist[pl.ds(base, 16)] + one
        return c

      lax.fori_loop(0, CHUNK // 16, edge_group, 0)

    def idx_fetch(g, s):
      pltpu.async_copy(pair_hbm.at[w, g], ring[s], isems[s])

    def idx_wait(s):
      pltpu.make_async_copy(pair_hbm.at[w, 0], ring[s], isems[s]).wait()

    def gather(s, b):
      pltpu.async_copy(x_hbm.at[ring[s].at[0]], bufs[b], gsems[b])

    def gather_wait(s, b):
      pltpu.make_async_copy(x_hbm.at[ring[s].at[0]], bufs[b], gsems[b]).wait()

    def scatter(s, b):
      pltpu.async_copy(bufs[b], agg_sh.at[ring[s].at[1]], ssems[b], add=True)

    def scatter_wait(s, b):
      pltpu.make_async_copy(bufs[b], agg_sh.at[ring[s].at[1]],
                            ssems[b]).wait()

    for s in range(IDXN - 1):
      idx_fetch(s, s)
    idx_wait(0)
    gather(0, 0)

    def outer(p, carry):
      for b in range(INNER):
        g = p * INNER + b
        s, s1, s3 = b % IDXN, (b + 1) % IDXN, (b + 3) % IDXN
        rb, rb1 = b % NBUF, (b + 1) % NBUF
        if b == 0:
          @pl.when(p >= 1)
          def _():
            scatter_wait(s3, rb1)
        else:
          scatter_wait(s3, rb1)
        if b == 0:
          idx_fetch(g + 3, s3)
        else:
          @pl.when(p < nouter - 1)
          def _():
            idx_fetch(g + 3, s3)
        if b == INNER - 1:
          @pl.when(p < nouter - 1)
          def _():
            idx_wait(s1)
            gather(s1, rb1)
        else:
          idx_wait(s1)
          gather(s1, rb1)
        hist_chunk(s)
        gather_wait(s, rb)
        scatter(s, rb)
      return carry

    lax.fori_loop(0, nouter, outer, 0)
    scatter_wait((nchunks - 1) % IDXN, (nchunks - 1) % NBUF)
    pltpu.sync_copy(hist, cnt_out.at[cid, sid])
    plsc.subcore_barrier()

    for q in range(8):
      rq = r0 + q * zch
      pltpu.sync_copy(agg_sh.at[pl.ds(rq, zch)], bufs[0])
      pltpu.sync_copy(bufs[0], agg_out.at[cid, pl.ds(rq, zch)])

  return agg_kernel


def _tc_body(agg_ref, cnt_ref, x_ref, wn_ref, ws_ref, b_ref, o_ref):
  blk, d = x_ref.shape
  nsub = blk // d
  agg = agg_ref[0] + agg_ref[1]
  cnt = jnp.sum(cnt_ref[...], axis=(0, 1)).reshape(nsub, d)
  recip = 1.0 / jnp.maximum(cnt, 1.0)
  dn = (((1,), (1,)), ((), ()))
  aggw = lax.dot_general(agg, wn_ref[...], dn,
                         preferred_element_type=jnp.float32)
  hs = lax.dot_general(x_ref[...], ws_ref[...], dn,
                       preferred_element_type=jnp.float32)
  ii = lax.broadcasted_iota(jnp.int32, (d, d), 0)
  jj = lax.broadcasted_iota(jnp.int32, (d, d), 1)
  eye = jnp.where(ii == jj, 1.0, 0.0)
  for a in range(nsub):
    dmat = eye * jnp.broadcast_to(recip[a:a + 1, :], (d, d))
    hn_a = lax.dot_general(dmat, aggw[a * d:(a + 1) * d, :],
                           (((1,), (0,)), ((), ())),
                           preferred_element_type=jnp.float32)
    h = hn_a + hs[a * d:(a + 1) * d, :] + b_ref[...]
    h = jnp.where(jnp.isfinite(h), h, 0.0)
    norm = jnp.sqrt(jnp.sum(h * h, axis=1, keepdims=True))
    o_ref[pl.ds(a * d, d), :] = h / jnp.maximum(norm, 1e-12)


def kernel(x, edge_index, W_neigh, W_self, bias):
  n, d = x.shape
  e = edge_index.shape[1]
  nw = NC * NS
  n_pad = 10240 if n == 10000 else ((n + 16 * d - 1) // (16 * d)) * (16 * d)
  egrain = nw * CHUNK * INNER
  e_pad = ((e + egrain - 1) // egrain) * egrain
  pad_rows = n_pad - n

  ei = edge_index
  if e_pad != e:
    spread = jnp.arange(e_pad - e, dtype=jnp.int32) % pad_rows
    ei = jnp.concatenate([ei, jnp.stack([spread, spread + n])], axis=1)
  nchunks = e_pad // nw // CHUNK
  pairs = ei.reshape(2, nw, nchunks, CHUNK).transpose(1, 2, 0, 3)

  zrow = jnp.zeros((CHUNK, d), jnp.float32)
  zcnt = jnp.zeros((n_pad,), jnp.float32)

  agg_parts, cnt_parts = _sc_aggregate(n_pad, e_pad, d)(
      x, pairs, zrow, zcnt)

  blk = n_pad // 4
  return pl.pallas_call(
      _tc_body,
      grid=(n_pad // blk,),
      in_specs=[
          pl.BlockSpec((NC, blk, d), lambda i: (0, i, 0)),
          pl.BlockSpec((NC, NS, blk), lambda i: (0, 0, i)),
          pl.BlockSpec((blk, d), lambda i: (i, 0)),
          pl.BlockSpec((d, d), lambda i: (0, 0)),
          pl.BlockSpec((d, d), lambda i: (0, 0)),
          pl.BlockSpec((1, d), lambda i: (0, 0)),
      ],
      out_specs=pl.BlockSpec((blk, d), lambda i: (i, 0)),
      out_shape=jax.ShapeDtypeStruct((n, d), jnp.float32),
  )(agg_parts, cnt_parts, x, W_neigh, W_self, bias.reshape(1, d))
